# baseline (speedup 1.0000x reference)
import functools

import jax
import jax.numpy as jnp
from jax import lax
from jax.experimental import pallas as pl
from jax.experimental.pallas import tpu as pltpu
from jax.experimental.pallas import tpu_sc as plsc

TOPK = 2
BT = 256
NLANE = 16


def _router_body(x_ref, wr_ref, br_ref, i1_ref, i2_ref, g1_ref, g2_ref):
    xb = x_ref[...]
    logits = jnp.dot(xb, wr_ref[...], preferred_element_type=jnp.float32)
    logits = logits + br_ref[...]
    bt, e = logits.shape
    iota_e = lax.broadcasted_iota(jnp.int32, (bt, e), 1)
    m1 = jnp.max(logits, axis=-1, keepdims=True)
    i1 = jnp.min(jnp.where(logits == m1, iota_e, e), axis=-1, keepdims=True)
    l2 = jnp.where(iota_e == i1, -jnp.inf, logits)
    m2 = jnp.max(l2, axis=-1, keepdims=True)
    i2 = jnp.min(jnp.where(l2 == m2, iota_e, e), axis=-1, keepdims=True)
    g1 = 1.0 / (1.0 + jnp.exp(m2 - m1))
    i1_ref[...] = i1
    i2_ref[...] = i2
    g1_ref[...] = g1
    g2_ref[...] = 1.0 - g1


def _router(x_flat, Wr, br, bt):
    T, D = x_flat.shape
    E = Wr.shape[1]
    o = jax.ShapeDtypeStruct((T, 1), jnp.int32)
    og = jax.ShapeDtypeStruct((T, 1), jnp.float32)
    return pl.pallas_call(
        _router_body,
        grid=(T // bt,),
        in_specs=[
            pl.BlockSpec((bt, D), lambda t: (t, 0)),
            pl.BlockSpec((D, E), lambda t: (0, 0)),
            pl.BlockSpec((1, E), lambda t: (0, 0)),
        ],
        out_specs=[pl.BlockSpec((bt, 1), lambda t: (t, 0))] * 4,
        out_shape=[o, o, og, og],
    )(x_flat, Wr, br.reshape(1, E))


def _wid():
    return lax.axis_index("s") * 2 + lax.axis_index("c")


def _make_count(T):
    Th = T // 2
    mesh = plsc.VectorSubcoreMesh(core_axis_name="c", subcore_axis_name="s")

    @functools.partial(
        pl.kernel,
        out_type=jax.ShapeDtypeStruct((32, NLANE), jnp.int32),
        mesh=mesh,
        compiler_params=pltpu.CompilerParams(needs_layout_passes=False),
        scratch_types=[
            pltpu.VMEM((Th,), jnp.int32),
            pltpu.VMEM((NLANE,), jnp.int32),
        ],
    )
    def count_k(ids_hbm, counts_hbm, ids_v, cnt_v):
        w = _wid()
        e = w // 4
        slot = (w // 2) % 2
        h = w % 2
        base = slot * T + h * Th
        pltpu.sync_copy(ids_hbm.at[pl.ds(base, Th)], ids_v)

        one = jnp.ones((NLANE,), jnp.int32)
        zero = jnp.zeros((NLANE,), jnp.int32)

        def step(i, acc):
            v = ids_v[pl.ds(i * NLANE, NLANE)]
            return acc + jnp.where(v == e, one, zero)

        acc = lax.fori_loop(0, Th // NLANE, step,
                            jnp.zeros((NLANE,), jnp.int32))
        total = jnp.sum(acc)
        cnt_v[...] = jnp.full((NLANE,), total, jnp.int32)
        pltpu.sync_copy(cnt_v, counts_hbm.at[w])

    return count_k


def _make_dispatch(T, E, SCHED, POSN):
    Th = T // 2
    NV = Th // NLANE
    NCHUNK = Th // 128
    DUMP1 = SCHED
    DUMP2 = TOPK * T
    mesh = plsc.VectorSubcoreMesh(core_axis_name="c", subcore_axis_name="s")

    @functools.partial(
        pl.kernel,
        out_type=[
            jax.ShapeDtypeStruct((SCHED + 64,), jnp.int32),
            jax.ShapeDtypeStruct((SCHED + 64,), jnp.float32),
            jax.ShapeDtypeStruct((POSN,), jnp.int32),
            jax.ShapeDtypeStruct((64,), jnp.int32),
        ],
        mesh=mesh,
        compiler_params=pltpu.CompilerParams(needs_layout_passes=False),
        scratch_types=[
            pltpu.VMEM((Th,), jnp.int32),
            pltpu.VMEM((Th,), jnp.float32),
            pltpu.VMEM((32, NLANE), jnp.int32),
            pltpu.VMEM((NCHUNK, 128), jnp.int32),
            pltpu.VMEM((NCHUNK, 128), jnp.int32),
            pltpu.VMEM((NCHUNK, 128), jnp.float32),
            pltpu.VMEM((NCHUNK, 128), jnp.int32),
            pltpu.VMEM((NCHUNK, 128), jnp.int32),
            pltpu.VMEM((64,), jnp.int32),
        ],
    )
    def dispatch_k(ids_hbm, gts_hbm, counts_hbm,
                   tok_hbm, gate_hbm, pos_hbm, be_hbm,
                   ids_v, gts_v, cnt_v, sidx_v, stok_v, sgt_v,
                   pidx_v, pval_v, be_v):
        w = _wid()
        e = w // 4
        slot = (w // 2) % 2
        h = w % 2
        base = slot * T + h * Th
        pltpu.sync_copy(ids_hbm.at[pl.ds(base, Th)], ids_v)
        pltpu.sync_copy(gts_hbm.at[pl.ds(base, Th)], gts_v)
        pltpu.sync_copy(counts_hbm, cnt_v)

        cw = [cnt_v[i, pl.ds(0, NLANE)][0] for i in range(32)]
        ce = [cw[4 * i] + cw[4 * i + 1] + cw[4 * i + 2] + cw[4 * i + 3]
              for i in range(E)]
        nb = [(c + BT - 1) // BT for c in ce]
        blkbase = []
        run = 0
        for i in range(E):
            blkbase.append(run)
            run = run + nb[i]
        row_base = jnp.int32(0)
        for i in range(E):
            row_base = row_base + jnp.where(e == i, blkbase[i] * BT, 0)
        k = slot * 2 + h
        for kp in range(4):
            sel = jnp.int32(0)
            for i in range(E):
                sel = sel + jnp.where(e == i, cw[4 * i + kp], 0)
            row_base = row_base + jnp.where(kp < k, sel, 0)

        @pl.when(w == 0)
        def _():
            onev = jnp.ones((NLANE,), jnp.int32)
            zerov = jnp.zeros((NLANE,), jnp.int32)
            for j4 in range(4):
                lanes = lax.iota(jnp.int32, NLANE) + j4 * NLANE
                bev = jnp.full((NLANE,), -1, jnp.int32)
                for i in range(E):
                    bev = bev + jnp.where(
                        lanes >= jnp.full((NLANE,), blkbase[i], jnp.int32),
                        onev, zerov)
                be_v[pl.ds(j4 * NLANE, NLANE)] = bev
            pltpu.sync_copy(be_v, be_hbm)

        iota16 = lax.iota(jnp.int32, NLANE)
        onev = jnp.ones((NLANE,), jnp.int32)
        zerov = jnp.zeros((NLANE,), jnp.int32)
        dump1 = jnp.full((NLANE,), DUMP1, jnp.int32)
        dump2 = jnp.full((NLANE,), DUMP2, jnp.int32)

        def step(iv, running):
            ids16 = ids_v[pl.ds(iv * NLANE, NLANE)]
            g16 = gts_v[pl.ds(iv * NLANE, NLANE)]
            m = ids16 == e
            mi = jnp.where(m, onev, zerov)
            pref = plsc.cumsum(mi)
            total = jnp.sum(mi)
            p = row_base + running + pref - 1
            tvec = h * Th + iv * NLANE + iota16
            j = iv // 8
            off = (iv % 8) * NLANE
            sidx_v[j, pl.ds(off, NLANE)] = jnp.where(m, p, dump1)
            stok_v[j, pl.ds(off, NLANE)] = tvec
            sgt_v[j, pl.ds(off, NLANE)] = g16
            pidx_v[j, pl.ds(off, NLANE)] = jnp.where(m, slot * T + tvec,
                                                     dump2)
            pval_v[j, pl.ds(off, NLANE)] = p
            return running + total

        lax.fori_loop(0, NV, step, jnp.int32(0))

        for j in range(NCHUNK):
            pltpu.sync_copy(stok_v.at[j], tok_hbm.at[sidx_v.at[j]])
            pltpu.sync_copy(sgt_v.at[j], gate_hbm.at[sidx_v.at[j]])
            pltpu.sync_copy(pval_v.at[j], pos_hbm.at[pidx_v.at[j]])

    return dispatch_k


def _make_gather(T, D, SCHED):
    rows_per = SCHED // 32
    CH = 64
    nch = rows_per // CH
    mesh = plsc.VectorSubcoreMesh(core_axis_name="c", subcore_axis_name="s")

    @functools.partial(
        pl.kernel,
        out_type=jax.ShapeDtypeStruct((SCHED, D), jnp.float32),
        mesh=mesh,
        compiler_params=pltpu.CompilerParams(needs_layout_passes=False),
        scratch_types=[
            pltpu.VMEM((CH,), jnp.int32),
            pltpu.VMEM((CH, D), jnp.float32),
            pltpu.SemaphoreType.DMA,
        ],
    )
    def gather_k(tok_hbm, x_hbm, xg_hbm, idx_v, rows_v, sem):
        w = _wid()
        for j in range(nch):
            base = w * rows_per + j * CH
            pltpu.sync_copy(tok_hbm.at[pl.ds(base, CH)], idx_v)
            lo = jnp.zeros((NLANE,), jnp.int32)
            hi = jnp.full((NLANE,), T - 1, jnp.int32)
            for kk in range(CH // NLANE):
                v = idx_v[pl.ds(kk * NLANE, NLANE)]
                v = jnp.minimum(jnp.maximum(v, lo), hi)
                idx_v[pl.ds(kk * NLANE, NLANE)] = v
            pltpu.async_copy(x_hbm.at[idx_v], rows_v, sem).wait()
            pltpu.sync_copy(rows_v, xg_hbm.at[pl.ds(base, CH)])

    return gather_k


def _gemm_body(be_ref, x_ref, w1_ref, b1_ref, w2_ref, b2_ref, g_ref,
               y_ref, acc_ref, *, n_hc, bh):
    xb = x_ref[...].astype(jnp.bfloat16)
    acc_ref[...] = jnp.zeros_like(acc_ref)
    for hc in range(n_hc):
        w1c = w1_ref[0, :, hc * bh:(hc + 1) * bh]
        h = jnp.dot(xb, w1c, preferred_element_type=jnp.float32)
        h = h + b1_ref[0, :, hc * bh:(hc + 1) * bh]
        h = h * (1.0 / (1.0 + jnp.exp(-h)))
        w2c = w2_ref[0, hc * bh:(hc + 1) * bh, :]
        acc_ref[...] += jnp.dot(h.astype(jnp.bfloat16), w2c,
                                preferred_element_type=jnp.float32)
    y_ref[...] = (acc_ref[...] + b2_ref[0]) * g_ref[...]


def _gemm(xg, W1b, b1, W2b, b2, row_gate, block_expert, nblk):
    _, D = xg.shape
    E, _, H = W1b.shape
    n_hc = 4
    grid_spec = pltpu.PrefetchScalarGridSpec(
        num_scalar_prefetch=1,
        grid=(nblk,),
        in_specs=[
            pl.BlockSpec((BT, D), lambda b, be: (b, 0)),
            pl.BlockSpec((1, D, H), lambda b, be: (be[b], 0, 0)),
            pl.BlockSpec((1, 1, H), lambda b, be: (be[b], 0, 0)),
            pl.BlockSpec((1, H, D), lambda b, be: (be[b], 0, 0)),
            pl.BlockSpec((1, 1, D), lambda b, be: (be[b], 0, 0)),
            pl.BlockSpec((BT, 1), lambda b, be: (b, 0)),
        ],
        out_specs=pl.BlockSpec((BT, D), lambda b, be: (b, 0)),
        scratch_shapes=[pltpu.VMEM((BT, D), jnp.float32)],
    )
    return pl.pallas_call(
        functools.partial(_gemm_body, n_hc=n_hc, bh=H // n_hc),
        grid_spec=grid_spec,
        out_shape=jax.ShapeDtypeStruct((nblk * BT, D), jnp.float32),
        compiler_params=pltpu.CompilerParams(vmem_limit_bytes=61_000_000),
    )(block_expert, xg, W1b, b1.reshape(E, 1, H), W2b, b2.reshape(E, 1, D),
      row_gate)


def _make_combine(T, D, SCHED):
    tok_per = T // 32
    CH = 32
    nch = tok_per // CH
    mesh = plsc.VectorSubcoreMesh(core_axis_name="c", subcore_axis_name="s")

    @functools.partial(
        pl.kernel,
        out_type=jax.ShapeDtypeStruct((T, D), jnp.float32),
        mesh=mesh,
        compiler_params=pltpu.CompilerParams(needs_layout_passes=False),
        scratch_types=[
            pltpu.VMEM((CH,), jnp.int32),
            pltpu.VMEM((CH,), jnp.int32),
            pltpu.VMEM((CH, D), jnp.float32),
            pltpu.VMEM((CH, D), jnp.float32),
            pltpu.SemaphoreType.DMA,
            pltpu.SemaphoreType.DMA,
        ],
    )
    def combine_k(pos_hbm, y_hbm, out_hbm, i0_v, i1_v, a_v, b_v, sem0, sem1):
        w = _wid()
        for j in range(nch):
            tbase = w * tok_per + j * CH
            pltpu.sync_copy(pos_hbm.at[pl.ds(tbase, CH)], i0_v)
            pltpu.sync_copy(pos_hbm.at[pl.ds(T + tbase, CH)], i1_v)
            c0 = pltpu.async_copy(y_hbm.at[i0_v], a_v, sem0)
            c1 = pltpu.async_copy(y_hbm.at[i1_v], b_v, sem1)
            c0.wait()
            c1.wait()
            for r in range(CH):
                def add_step(i, _):
                    sl = pl.ds(i * NLANE, NLANE)
                    a_v[r, sl] = a_v[r, sl] + b_v[r, sl]
                    return 0
                lax.fori_loop(0, D // NLANE, add_step, 0)
            pltpu.sync_copy(a_v, out_hbm.at[pl.ds(tbase, CH)])

    return combine_k


@jax.jit
def kernel(x, Wr, br, W1, b1, W2, b2):
    B, S, D = x.shape
    E = Wr.shape[1]
    H = W1.shape[2]
    x_flat = x.reshape(-1, D)
    T = x_flat.shape[0]
    SCHED = (T * TOPK // BT + E) * BT
    POSN = TOPK * T + 64
    nblk = SCHED // BT

    i1, i2, g1, g2 = _router(x_flat, Wr, br, min(512, T))
    ids_all = jnp.concatenate([i1.reshape(-1), i2.reshape(-1)])
    gts_all = jnp.concatenate([g1.reshape(-1), g2.reshape(-1)])

    counts = _make_count(T)(ids_all)
    src_tok, row_gate, pos, block_expert = _make_dispatch(T, E, SCHED, POSN)(
        ids_all, gts_all, counts)
    xg = _make_gather(T, D, SCHED)(src_tok, x_flat)
    y = _gemm(xg, W1.astype(jnp.bfloat16), b1, W2.astype(jnp.bfloat16), b2,
              row_gate[:SCHED].reshape(SCHED, 1), block_expert[:nblk], nblk)
    out = _make_combine(T, D, SCHED)(pos, y)
    return out.reshape(x.shape)


# speedup vs baseline: 9.8682x; 9.8682x over previous
import functools

import jax
import jax.numpy as jnp
from jax import lax
from jax.experimental import pallas as pl
from jax.experimental.pallas import tpu as pltpu
from jax.experimental.pallas import tpu_sc as plsc

TOPK = 2
BT = 256
NLANE = 16


def _router_body(x_ref, wr_ref, br_ref, i1_ref, i2_ref, g1_ref, g2_ref):
    xb = x_ref[...]
    logits = jnp.dot(xb, wr_ref[...], preferred_element_type=jnp.float32)
    logits = logits + br_ref[...]
    bt, e = logits.shape
    iota_e = lax.broadcasted_iota(jnp.int32, (bt, e), 1)
    m1 = jnp.max(logits, axis=-1, keepdims=True)
    i1 = jnp.min(jnp.where(logits == m1, iota_e, e), axis=-1, keepdims=True)
    l2 = jnp.where(iota_e == i1, -jnp.inf, logits)
    m2 = jnp.max(l2, axis=-1, keepdims=True)
    i2 = jnp.min(jnp.where(l2 == m2, iota_e, e), axis=-1, keepdims=True)
    g1 = 1.0 / (1.0 + jnp.exp(m2 - m1))
    i1_ref[...] = i1
    i2_ref[...] = i2
    g1_ref[...] = g1
    g2_ref[...] = 1.0 - g1


def _router(x_flat, Wr, br, bt):
    T, D = x_flat.shape
    E = Wr.shape[1]
    o = jax.ShapeDtypeStruct((T, 1), jnp.int32)
    og = jax.ShapeDtypeStruct((T, 1), jnp.float32)
    return pl.pallas_call(
        _router_body,
        grid=(T // bt,),
        in_specs=[
            pl.BlockSpec((bt, D), lambda t: (t, 0)),
            pl.BlockSpec((D, E), lambda t: (0, 0)),
            pl.BlockSpec((1, E), lambda t: (0, 0)),
        ],
        out_specs=[pl.BlockSpec((bt, 1), lambda t: (t, 0))] * 4,
        out_shape=[o, o, og, og],
    )(x_flat, Wr, br.reshape(1, E))


def _wid():
    return lax.axis_index("s") * 2 + lax.axis_index("c")


def _make_count(T):
    Th = T // 2
    mesh = plsc.VectorSubcoreMesh(core_axis_name="c", subcore_axis_name="s")

    @functools.partial(
        pl.kernel,
        out_type=jax.ShapeDtypeStruct((32, NLANE), jnp.int32),
        mesh=mesh,
        compiler_params=pltpu.CompilerParams(needs_layout_passes=False, use_tc_tiling_on_sc=False),
        scratch_types=[
            pltpu.VMEM((Th,), jnp.int32),
            pltpu.VMEM((NLANE,), jnp.int32),
        ],
    )
    def count_k(ids_hbm, counts_hbm, ids_v, cnt_v):
        w = _wid()
        e = w // 4
        slot = (w // 2) % 2
        h = w % 2
        base = slot * T + h * Th
        pltpu.sync_copy(ids_hbm.at[pl.ds(base, Th)], ids_v)

        one = jnp.ones((NLANE,), jnp.int32)
        zero = jnp.zeros((NLANE,), jnp.int32)

        def step(i, acc):
            v = ids_v[pl.ds(i * NLANE, NLANE)]
            return acc + jnp.where(v == e, one, zero)

        acc = lax.fori_loop(0, Th // NLANE, step,
                            jnp.zeros((NLANE,), jnp.int32))
        total = jnp.sum(acc)
        cnt_v[...] = jnp.full((NLANE,), total, jnp.int32)
        pltpu.sync_copy(cnt_v, counts_hbm.at[w])

    return count_k


def _make_dispatch(T, E, SCHED, POSN):
    Th = T // 2
    NV = Th // NLANE
    NCHUNK = Th // 128
    DUMP1 = SCHED
    DUMP2 = TOPK * T
    mesh = plsc.VectorSubcoreMesh(core_axis_name="c", subcore_axis_name="s")

    @functools.partial(
        pl.kernel,
        out_type=[
            jax.ShapeDtypeStruct((SCHED + 64, NLANE), jnp.float32),
            jax.ShapeDtypeStruct((POSN, NLANE), jnp.int32),
            jax.ShapeDtypeStruct((64,), jnp.int32),
        ],
        mesh=mesh,
        compiler_params=pltpu.CompilerParams(needs_layout_passes=False, use_tc_tiling_on_sc=False),
        scratch_types=[
            pltpu.VMEM((Th,), jnp.int32),
            pltpu.VMEM((Th,), jnp.float32),
            pltpu.VMEM((32, NLANE), jnp.int32),
            pltpu.VMEM((NCHUNK, 128), jnp.int32),
            pltpu.VMEM((NCHUNK, 128, NLANE), jnp.float32),
            pltpu.VMEM((NCHUNK, 128), jnp.int32),
            pltpu.VMEM((NCHUNK, 128, NLANE), jnp.int32),
            pltpu.VMEM((64,), jnp.int32),
        ],
    )
    def dispatch_k(ids_hbm, gts_hbm, counts_hbm,
                   meta_hbm, pos_hbm, be_hbm,
                   ids_v, gts_v, cnt_v, sidx_v, sval_v,
                   pidx_v, pval_v, be_v):
        w = _wid()
        e = w // 4
        slot = (w // 2) % 2
        h = w % 2
        base = slot * T + h * Th
        pltpu.sync_copy(ids_hbm.at[pl.ds(base, Th)], ids_v)
        pltpu.sync_copy(gts_hbm.at[pl.ds(base, Th)], gts_v)
        pltpu.sync_copy(counts_hbm, cnt_v)

        cw = [cnt_v[i, pl.ds(0, NLANE)][0] for i in range(32)]
        ce = [cw[4 * i] + cw[4 * i + 1] + cw[4 * i + 2] + cw[4 * i + 3]
              for i in range(E)]
        nb = [(c + BT - 1) // BT for c in ce]
        blkbase = []
        run = 0
        for i in range(E):
            blkbase.append(run)
            run = run + nb[i]
        row_base = jnp.int32(0)
        for i in range(E):
            row_base = row_base + jnp.where(e == i, blkbase[i] * BT, 0)
        k = slot * 2 + h
        for kp in range(4):
            sel = jnp.int32(0)
            for i in range(E):
                sel = sel + jnp.where(e == i, cw[4 * i + kp], 0)
            row_base = row_base + jnp.where(kp < k, sel, 0)

        @pl.when(w == 0)
        def _():
            onev = jnp.ones((NLANE,), jnp.int32)
            zerov = jnp.zeros((NLANE,), jnp.int32)
            for j4 in range(4):
                lanes = lax.iota(jnp.int32, NLANE) + j4 * NLANE
                bev = jnp.full((NLANE,), -1, jnp.int32)
                for i in range(E):
                    bev = bev + jnp.where(
                        lanes >= jnp.full((NLANE,), blkbase[i], jnp.int32),
                        onev, zerov)
                be_v[pl.ds(j4 * NLANE, NLANE)] = bev
            pltpu.sync_copy(be_v, be_hbm)

        iota16 = lax.iota(jnp.int32, NLANE)
        onev = jnp.ones((NLANE,), jnp.int32)
        zerov = jnp.zeros((NLANE,), jnp.int32)
        dump1 = jnp.full((NLANE,), DUMP1, jnp.int32)
        dump2 = jnp.full((NLANE,), DUMP2, jnp.int32)

        def step(iv, running):
            ids16 = ids_v[pl.ds(iv * NLANE, NLANE)]
            g16 = gts_v[pl.ds(iv * NLANE, NLANE)]
            m = ids16 == e
            mi = jnp.where(m, onev, zerov)
            pref = plsc.cumsum(mi)
            total = jnp.sum(mi)
            p = row_base + running + pref - 1
            tvec = h * Th + iv * NLANE + iota16
            j = iv // 8
            off = (iv % 8) * NLANE
            rows = off + iota16
            jv = jnp.full((NLANE,), j, jnp.int32)
            sidx_v[j, pl.ds(off, NLANE)] = jnp.where(m, p, dump1)
            plsc.store_scatter(sval_v, [jv, rows, zerov],
                               tvec.astype(jnp.float32))
            plsc.store_scatter(sval_v, [jv, rows, onev], g16)
            pidx_v[j, pl.ds(off, NLANE)] = jnp.where(m, slot * T + tvec,
                                                     dump2)
            plsc.store_scatter(pval_v, [jv, rows, zerov], p)
            return running + total

        lax.fori_loop(0, NV, step, jnp.int32(0))

        for j in range(NCHUNK):
            pltpu.sync_copy(sval_v.at[j], meta_hbm.at[sidx_v.at[j]])
            pltpu.sync_copy(pval_v.at[j], pos_hbm.at[pidx_v.at[j]])

    return dispatch_k


def _make_gather(T, D, SCHED):
    rows_per = SCHED // 32
    CH = 64
    nch = rows_per // CH
    mesh = plsc.VectorSubcoreMesh(core_axis_name="c", subcore_axis_name="s")

    @functools.partial(
        pl.kernel,
        out_type=jax.ShapeDtypeStruct((SCHED, D), jnp.float32),
        mesh=mesh,
        compiler_params=pltpu.CompilerParams(needs_layout_passes=False, use_tc_tiling_on_sc=False),
        scratch_types=[
            pltpu.VMEM((CH, NLANE), jnp.float32),
            pltpu.VMEM((CH,), jnp.int32),
            pltpu.VMEM((CH, D), jnp.float32),
            pltpu.SemaphoreType.DMA,
        ],
    )
    def gather_k(meta_hbm, x_hbm, xg_hbm, meta_v, idx_v, rows_v, sem):
        w = _wid()
        iota16 = lax.iota(jnp.int32, NLANE)
        lo = jnp.zeros((NLANE,), jnp.int32)
        hi = jnp.full((NLANE,), T - 1, jnp.int32)
        zerov = jnp.zeros((NLANE,), jnp.int32)
        for j in range(nch):
            base = w * rows_per + j * CH
            pltpu.sync_copy(meta_hbm.at[pl.ds(base, CH)], meta_v)
            for kk in range(CH // NLANE):
                rows = iota16 + kk * NLANE
                v = plsc.load_gather(meta_v, [rows, zerov]).astype(jnp.int32)
                v = jnp.minimum(jnp.maximum(v, lo), hi)
                idx_v[pl.ds(kk * NLANE, NLANE)] = v
            pltpu.async_copy(x_hbm.at[idx_v], rows_v, sem).wait()
            pltpu.sync_copy(rows_v, xg_hbm.at[pl.ds(base, CH)])

    return gather_k


def _gemm_body(be_ref, x_ref, w1_ref, b1_ref, w2_ref, b2_ref, meta_ref,
               y_ref, acc_ref, *, n_hc, bh):
    xb = x_ref[...].astype(jnp.bfloat16)
    acc_ref[...] = jnp.zeros_like(acc_ref)
    for hc in range(n_hc):
        w1c = w1_ref[0, :, hc * bh:(hc + 1) * bh]
        h = jnp.dot(xb, w1c, preferred_element_type=jnp.float32)
        h = h + b1_ref[0, :, hc * bh:(hc + 1) * bh]
        h = h * (1.0 / (1.0 + jnp.exp(-h)))
        w2c = w2_ref[0, hc * bh:(hc + 1) * bh, :]
        acc_ref[...] += jnp.dot(h.astype(jnp.bfloat16), w2c,
                                preferred_element_type=jnp.float32)
    g = meta_ref[...][:, 1:2]
    y_ref[...] = (acc_ref[...] + b2_ref[0]) * g


def _gemm(xg, W1b, b1, W2b, b2, meta, block_expert, nblk):
    _, D = xg.shape
    E, _, H = W1b.shape
    n_hc = 4
    grid_spec = pltpu.PrefetchScalarGridSpec(
        num_scalar_prefetch=1,
        grid=(nblk,),
        in_specs=[
            pl.BlockSpec((BT, D), lambda b, be: (b, 0)),
            pl.BlockSpec((1, D, H), lambda b, be: (be[b], 0, 0)),
            pl.BlockSpec((1, 1, H), lambda b, be: (be[b], 0, 0)),
            pl.BlockSpec((1, H, D), lambda b, be: (be[b], 0, 0)),
            pl.BlockSpec((1, 1, D), lambda b, be: (be[b], 0, 0)),
            pl.BlockSpec((BT, NLANE), lambda b, be: (b, 0)),
        ],
        out_specs=pl.BlockSpec((BT, D), lambda b, be: (b, 0)),
        scratch_shapes=[pltpu.VMEM((BT, D), jnp.float32)],
    )
    return pl.pallas_call(
        functools.partial(_gemm_body, n_hc=n_hc, bh=H // n_hc),
        grid_spec=grid_spec,
        out_shape=jax.ShapeDtypeStruct((nblk * BT, D), jnp.float32),
        compiler_params=pltpu.CompilerParams(vmem_limit_bytes=61_000_000),
    )(block_expert, xg, W1b, b1.reshape(E, 1, H), W2b, b2.reshape(E, 1, D),
      meta)


def _make_combine(T, D, SCHED):
    tok_per = T // 32
    CH = 32
    nch = tok_per // CH
    mesh = plsc.VectorSubcoreMesh(core_axis_name="c", subcore_axis_name="s")

    @functools.partial(
        pl.kernel,
        out_type=jax.ShapeDtypeStruct((T, D), jnp.float32),
        mesh=mesh,
        compiler_params=pltpu.CompilerParams(needs_layout_passes=False, use_tc_tiling_on_sc=False),
        scratch_types=[
            pltpu.VMEM((CH, NLANE), jnp.int32),
            pltpu.VMEM((CH, NLANE), jnp.int32),
            pltpu.VMEM((CH,), jnp.int32),
            pltpu.VMEM((CH,), jnp.int32),
            pltpu.VMEM((CH, D), jnp.float32),
            pltpu.VMEM((CH, D), jnp.float32),
            pltpu.SemaphoreType.DMA,
            pltpu.SemaphoreType.DMA,
        ],
    )
    def combine_k(pos_hbm, y_hbm, out_hbm, p0_v, p1_v, i0_v, i1_v,
                  a_v, b_v, sem0, sem1):
        w = _wid()
        iota16 = lax.iota(jnp.int32, NLANE)
        zerov = jnp.zeros((NLANE,), jnp.int32)
        for j in range(nch):
            tbase = w * tok_per + j * CH
            pltpu.sync_copy(pos_hbm.at[pl.ds(tbase, CH)], p0_v)
            pltpu.sync_copy(pos_hbm.at[pl.ds(T + tbase, CH)], p1_v)
            for kk in range(CH // NLANE):
                rows = iota16 + kk * NLANE
                i0_v[pl.ds(kk * NLANE, NLANE)] = plsc.load_gather(
                    p0_v, [rows, zerov])
                i1_v[pl.ds(kk * NLANE, NLANE)] = plsc.load_gather(
                    p1_v, [rows, zerov])
            c0 = pltpu.async_copy(y_hbm.at[i0_v], a_v, sem0)
            c1 = pltpu.async_copy(y_hbm.at[i1_v], b_v, sem1)
            c0.wait()
            c1.wait()
            for r in range(CH):
                def add_step(i, _):
                    sl = pl.ds(i * NLANE, NLANE)
                    a_v[r, sl] = a_v[r, sl] + b_v[r, sl]
                    return 0
                lax.fori_loop(0, D // NLANE, add_step, 0)
            pltpu.sync_copy(a_v, out_hbm.at[pl.ds(tbase, CH)])

    return combine_k


@jax.jit
def kernel(x, Wr, br, W1, b1, W2, b2):
    B, S, D = x.shape
    E = Wr.shape[1]
    H = W1.shape[2]
    x_flat = x.reshape(-1, D)
    T = x_flat.shape[0]
    SCHED = (T * TOPK // BT + E) * BT
    POSN = TOPK * T + 64
    nblk = SCHED // BT

    i1, i2, g1, g2 = _router(x_flat, Wr, br, min(512, T))
    ids_all = jnp.concatenate([i1.reshape(-1), i2.reshape(-1)])
    gts_all = jnp.concatenate([g1.reshape(-1), g2.reshape(-1)])

    counts = _make_count(T)(ids_all)
    meta, pos, block_expert = _make_dispatch(T, E, SCHED, POSN)(
        ids_all, gts_all, counts)
    xg = _make_gather(T, D, SCHED)(meta, x_flat)
    y = _gemm(xg, W1.astype(jnp.bfloat16), b1, W2.astype(jnp.bfloat16), b2,
              meta[:SCHED], block_expert[:nblk], nblk)
    out = _make_combine(T, D, SCHED)(pos, y)
    return out.reshape(x.shape)


# speedup vs baseline: 9.8792x; 1.0011x over previous
import functools

import jax
import jax.numpy as jnp
from jax import lax
from jax.experimental import pallas as pl
from jax.experimental.pallas import tpu as pltpu
from jax.experimental.pallas import tpu_sc as plsc

TOPK = 2
BT = 256
NLANE = 16


def _router_body(x_ref, wr_ref, br_ref, i1_ref, i2_ref, g1_ref, g2_ref):
    xb = x_ref[...]
    logits = jnp.dot(xb, wr_ref[...], preferred_element_type=jnp.float32)
    logits = logits + br_ref[...]
    bt, e = logits.shape
    iota_e = lax.broadcasted_iota(jnp.int32, (bt, e), 1)
    m1 = jnp.max(logits, axis=-1, keepdims=True)
    i1 = jnp.min(jnp.where(logits == m1, iota_e, e), axis=-1, keepdims=True)
    l2 = jnp.where(iota_e == i1, -jnp.inf, logits)
    m2 = jnp.max(l2, axis=-1, keepdims=True)
    i2 = jnp.min(jnp.where(l2 == m2, iota_e, e), axis=-1, keepdims=True)
    g1 = 1.0 / (1.0 + jnp.exp(m2 - m1))
    i1_ref[...] = i1
    i2_ref[...] = i2
    g1_ref[...] = g1
    g2_ref[...] = 1.0 - g1


def _router(x_flat, Wr, br, bt):
    T, D = x_flat.shape
    E = Wr.shape[1]
    o = jax.ShapeDtypeStruct((T, 1), jnp.int32)
    og = jax.ShapeDtypeStruct((T, 1), jnp.float32)
    return pl.pallas_call(
        _router_body,
        grid=(T // bt,),
        in_specs=[
            pl.BlockSpec((bt, D), lambda t: (t, 0)),
            pl.BlockSpec((D, E), lambda t: (0, 0)),
            pl.BlockSpec((1, E), lambda t: (0, 0)),
        ],
        out_specs=[pl.BlockSpec((bt, 1), lambda t: (t, 0))] * 4,
        out_shape=[o, o, og, og],
    )(x_flat, Wr, br.reshape(1, E))


def _wid():
    return lax.axis_index("s") * 2 + lax.axis_index("c")


def _make_count(T):
    Th = T // 2
    mesh = plsc.VectorSubcoreMesh(core_axis_name="c", subcore_axis_name="s")

    @functools.partial(
        pl.kernel,
        out_type=jax.ShapeDtypeStruct((32, NLANE), jnp.int32),
        mesh=mesh,
        compiler_params=pltpu.CompilerParams(needs_layout_passes=False, use_tc_tiling_on_sc=False),
        scratch_types=[
            pltpu.VMEM((Th,), jnp.int32),
            pltpu.VMEM((NLANE,), jnp.int32),
        ],
    )
    def count_k(ids_hbm, counts_hbm, ids_v, cnt_v):
        w = _wid()
        e = w // 4
        slot = (w // 2) % 2
        h = w % 2
        base = slot * T + h * Th
        pltpu.sync_copy(ids_hbm.at[pl.ds(base, Th)], ids_v)

        one = jnp.ones((NLANE,), jnp.int32)
        zero = jnp.zeros((NLANE,), jnp.int32)

        def step(i, acc):
            v = ids_v[pl.ds(i * NLANE, NLANE)]
            return acc + jnp.where(v == e, one, zero)

        acc = lax.fori_loop(0, Th // NLANE, step,
                            jnp.zeros((NLANE,), jnp.int32))
        total = jnp.sum(acc)
        cnt_v[...] = jnp.full((NLANE,), total, jnp.int32)
        pltpu.sync_copy(cnt_v, counts_hbm.at[w])

    return count_k


def _make_dispatch(T, E, SCHED, POSN):
    Th = T // 2
    NV = Th // NLANE
    NCHUNK = Th // 128
    DUMP1 = SCHED
    DUMP2 = TOPK * T
    mesh = plsc.VectorSubcoreMesh(core_axis_name="c", subcore_axis_name="s")

    @functools.partial(
        pl.kernel,
        out_type=[
            jax.ShapeDtypeStruct((SCHED + 64, NLANE), jnp.float32),
            jax.ShapeDtypeStruct((POSN, NLANE), jnp.int32),
            jax.ShapeDtypeStruct((64,), jnp.int32),
        ],
        mesh=mesh,
        compiler_params=pltpu.CompilerParams(needs_layout_passes=False, use_tc_tiling_on_sc=False),
        scratch_types=[
            pltpu.VMEM((Th,), jnp.int32),
            pltpu.VMEM((Th,), jnp.float32),
            pltpu.VMEM((32, NLANE), jnp.int32),
            pltpu.VMEM((NCHUNK, 128), jnp.int32),
            pltpu.VMEM((NCHUNK, 128, NLANE), jnp.float32),
            pltpu.VMEM((NCHUNK, 128), jnp.int32),
            pltpu.VMEM((NCHUNK, 128, NLANE), jnp.int32),
            pltpu.VMEM((64,), jnp.int32),
            pltpu.SemaphoreType.DMA,
            pltpu.SemaphoreType.DMA,
        ],
    )
    def dispatch_k(ids_hbm, gts_hbm, counts_hbm,
                   meta_hbm, pos_hbm, be_hbm,
                   ids_v, gts_v, cnt_v, sidx_v, sval_v,
                   pidx_v, pval_v, be_v, ssem, psem):
        w = _wid()
        e = w // 4
        slot = (w // 2) % 2
        h = w % 2
        base = slot * T + h * Th
        pltpu.sync_copy(ids_hbm.at[pl.ds(base, Th)], ids_v)
        pltpu.sync_copy(gts_hbm.at[pl.ds(base, Th)], gts_v)
        pltpu.sync_copy(counts_hbm, cnt_v)

        cw = [cnt_v[i, pl.ds(0, NLANE)][0] for i in range(32)]
        ce = [cw[4 * i] + cw[4 * i + 1] + cw[4 * i + 2] + cw[4 * i + 3]
              for i in range(E)]
        nb = [(c + BT - 1) // BT for c in ce]
        blkbase = []
        run = 0
        for i in range(E):
            blkbase.append(run)
            run = run + nb[i]
        row_base = jnp.int32(0)
        for i in range(E):
            row_base = row_base + jnp.where(e == i, blkbase[i] * BT, 0)
        k = slot * 2 + h
        for kp in range(4):
            sel = jnp.int32(0)
            for i in range(E):
                sel = sel + jnp.where(e == i, cw[4 * i + kp], 0)
            row_base = row_base + jnp.where(kp < k, sel, 0)

        @pl.when(w == 0)
        def _():
            onev = jnp.ones((NLANE,), jnp.int32)
            zerov = jnp.zeros((NLANE,), jnp.int32)
            for j4 in range(4):
                lanes = lax.iota(jnp.int32, NLANE) + j4 * NLANE
                bev = jnp.full((NLANE,), -1, jnp.int32)
                for i in range(E):
                    bev = bev + jnp.where(
                        lanes >= jnp.full((NLANE,), blkbase[i], jnp.int32),
                        onev, zerov)
                be_v[pl.ds(j4 * NLANE, NLANE)] = bev
            pltpu.sync_copy(be_v, be_hbm)

        iota16 = lax.iota(jnp.int32, NLANE)
        onev = jnp.ones((NLANE,), jnp.int32)
        zerov = jnp.zeros((NLANE,), jnp.int32)
        dump1 = jnp.full((NLANE,), DUMP1, jnp.int32)
        dump2 = jnp.full((NLANE,), DUMP2, jnp.int32)

        def step(iv, running):
            ids16 = ids_v[pl.ds(iv * NLANE, NLANE)]
            g16 = gts_v[pl.ds(iv * NLANE, NLANE)]
            m = ids16 == e
            mi = jnp.where(m, onev, zerov)
            pref = plsc.cumsum(mi)
            total = jnp.sum(mi)
            p = row_base + running + pref - 1
            tvec = h * Th + iv * NLANE + iota16
            j = iv // 8
            off = (iv % 8) * NLANE
            rows = off + iota16
            jv = jnp.full((NLANE,), j, jnp.int32)
            sidx_v[j, pl.ds(off, NLANE)] = jnp.where(m, p, dump1)
            plsc.store_scatter(sval_v, [jv, rows, zerov],
                               tvec.astype(jnp.float32))
            plsc.store_scatter(sval_v, [jv, rows, onev], g16)
            pidx_v[j, pl.ds(off, NLANE)] = jnp.where(m, slot * T + tvec,
                                                     dump2)
            plsc.store_scatter(pval_v, [jv, rows, zerov], p)
            return running + total

        lax.fori_loop(0, NV, step, jnp.int32(0))

        descs = []
        for j in range(NCHUNK):
            descs.append(pltpu.async_copy(
                sval_v.at[j], meta_hbm.at[sidx_v.at[j]], ssem))
            descs.append(pltpu.async_copy(
                pval_v.at[j], pos_hbm.at[pidx_v.at[j]], psem))
        for d in descs:
            d.wait()

    return dispatch_k


def _make_gather(T, D, SCHED):
    rows_per = SCHED // 32
    CH = 40
    nch = rows_per // CH
    mesh = plsc.VectorSubcoreMesh(core_axis_name="c", subcore_axis_name="s")

    @functools.partial(
        pl.kernel,
        out_type=jax.ShapeDtypeStruct((SCHED, D), jnp.float32),
        mesh=mesh,
        compiler_params=pltpu.CompilerParams(needs_layout_passes=False, use_tc_tiling_on_sc=False),
        scratch_types=[
            pltpu.VMEM((rows_per, NLANE), jnp.float32),
            pltpu.VMEM((rows_per,), jnp.int32),
            pltpu.VMEM((CH, D), jnp.float32),
            pltpu.VMEM((CH, D), jnp.float32),
            pltpu.SemaphoreType.DMA,
            pltpu.SemaphoreType.DMA,
            pltpu.SemaphoreType.DMA,
            pltpu.SemaphoreType.DMA,
        ],
    )
    def gather_k(meta_hbm, x_hbm, xg_hbm, meta_v, idx_v, rows_a, rows_b,
                 gsem0, gsem1, wsem0, wsem1):
        w = _wid()
        base0 = w * rows_per
        pltpu.sync_copy(meta_hbm.at[pl.ds(base0, rows_per)], meta_v)
        iota16 = lax.iota(jnp.int32, NLANE)
        lo = jnp.zeros((NLANE,), jnp.int32)
        hi = jnp.full((NLANE,), T - 1, jnp.int32)
        zerov = jnp.zeros((NLANE,), jnp.int32)
        for kk in range(rows_per // NLANE):
            rows = iota16 + kk * NLANE
            v = plsc.load_gather(meta_v, [rows, zerov]).astype(jnp.int32)
            v = jnp.minimum(jnp.maximum(v, lo), hi)
            idx_v[pl.ds(kk * NLANE, NLANE)] = v

        bufs = [rows_a, rows_b]
        gsems = [gsem0, gsem1]
        wsems = [wsem0, wsem1]
        gd = [None, None]
        wd = [None, None]
        for j in range(nch):
            b = j % 2
            if wd[b] is not None:
                wd[b].wait()
            gd[b] = pltpu.async_copy(
                x_hbm.at[idx_v.at[pl.ds(j * CH, CH)]], bufs[b], gsems[b])
            gd[b].wait()
            wd[b] = pltpu.async_copy(
                bufs[b], xg_hbm.at[pl.ds(base0 + j * CH, CH)], wsems[b])
        for b in range(2):
            if wd[b] is not None:
                wd[b].wait()

    return gather_k


def _gemm_body(be_ref, x_ref, w1_ref, b1_ref, w2_ref, b2_ref, meta_ref,
               y_ref, acc_ref, *, n_hc, bh):
    xb = x_ref[...].astype(jnp.bfloat16)
    acc_ref[...] = jnp.zeros_like(acc_ref)
    for hc in range(n_hc):
        w1c = w1_ref[0, :, hc * bh:(hc + 1) * bh]
        h = jnp.dot(xb, w1c, preferred_element_type=jnp.float32)
        h = h + b1_ref[0, :, hc * bh:(hc + 1) * bh]
        h = h * (1.0 / (1.0 + jnp.exp(-h)))
        w2c = w2_ref[0, hc * bh:(hc + 1) * bh, :]
        acc_ref[...] += jnp.dot(h.astype(jnp.bfloat16), w2c,
                                preferred_element_type=jnp.float32)
    g = meta_ref[...][:, 1:2]
    y_ref[...] = (acc_ref[...] + b2_ref[0]) * g


def _gemm(xg, W1b, b1, W2b, b2, meta, block_expert, nblk):
    _, D = xg.shape
    E, _, H = W1b.shape
    n_hc = 4
    grid_spec = pltpu.PrefetchScalarGridSpec(
        num_scalar_prefetch=1,
        grid=(nblk,),
        in_specs=[
            pl.BlockSpec((BT, D), lambda b, be: (b, 0)),
            pl.BlockSpec((1, D, H), lambda b, be: (be[b], 0, 0)),
            pl.BlockSpec((1, 1, H), lambda b, be: (be[b], 0, 0)),
            pl.BlockSpec((1, H, D), lambda b, be: (be[b], 0, 0)),
            pl.BlockSpec((1, 1, D), lambda b, be: (be[b], 0, 0)),
            pl.BlockSpec((BT, NLANE), lambda b, be: (b, 0)),
        ],
        out_specs=pl.BlockSpec((BT, D), lambda b, be: (b, 0)),
        scratch_shapes=[pltpu.VMEM((BT, D), jnp.float32)],
    )
    return pl.pallas_call(
        functools.partial(_gemm_body, n_hc=n_hc, bh=H // n_hc),
        grid_spec=grid_spec,
        out_shape=jax.ShapeDtypeStruct((nblk * BT, D), jnp.float32),
        compiler_params=pltpu.CompilerParams(vmem_limit_bytes=61_000_000),
    )(block_expert, xg, W1b, b1.reshape(E, 1, H), W2b, b2.reshape(E, 1, D),
      meta)


def _make_combine(T, D, SCHED):
    tok_per = T // 32
    CH = 32
    nch = tok_per // CH
    mesh = plsc.VectorSubcoreMesh(core_axis_name="c", subcore_axis_name="s")

    @functools.partial(
        pl.kernel,
        out_type=jax.ShapeDtypeStruct((T, D), jnp.float32),
        mesh=mesh,
        compiler_params=pltpu.CompilerParams(needs_layout_passes=False, use_tc_tiling_on_sc=False),
        scratch_types=[
            pltpu.VMEM((CH, NLANE), jnp.int32),
            pltpu.VMEM((CH, NLANE), jnp.int32),
            pltpu.VMEM((CH,), jnp.int32),
            pltpu.VMEM((CH,), jnp.int32),
            pltpu.VMEM((CH, D), jnp.float32),
            pltpu.VMEM((CH, D), jnp.float32),
            pltpu.SemaphoreType.DMA,
            pltpu.SemaphoreType.DMA,
        ],
    )
    def combine_k(pos_hbm, y_hbm, out_hbm, p0_v, p1_v, i0_v, i1_v,
                  a_v, b_v, sem0, sem1):
        w = _wid()
        iota16 = lax.iota(jnp.int32, NLANE)
        zerov = jnp.zeros((NLANE,), jnp.int32)
        for j in range(nch):
            tbase = w * tok_per + j * CH
            pltpu.sync_copy(pos_hbm.at[pl.ds(tbase, CH)], p0_v)
            pltpu.sync_copy(pos_hbm.at[pl.ds(T + tbase, CH)], p1_v)
            for kk in range(CH // NLANE):
                rows = iota16 + kk * NLANE
                i0_v[pl.ds(kk * NLANE, NLANE)] = plsc.load_gather(
                    p0_v, [rows, zerov])
                i1_v[pl.ds(kk * NLANE, NLANE)] = plsc.load_gather(
                    p1_v, [rows, zerov])
            c0 = pltpu.async_copy(y_hbm.at[i0_v], a_v, sem0)
            c1 = pltpu.async_copy(y_hbm.at[i1_v], b_v, sem1)
            c0.wait()
            c1.wait()
            for r in range(CH):
                def add_step(i, _):
                    sl = pl.ds(i * NLANE, NLANE)
                    a_v[r, sl] = a_v[r, sl] + b_v[r, sl]
                    return 0
                lax.fori_loop(0, D // NLANE, add_step, 0)
            pltpu.sync_copy(a_v, out_hbm.at[pl.ds(tbase, CH)])

    return combine_k


@jax.jit
def kernel(x, Wr, br, W1, b1, W2, b2):
    B, S, D = x.shape
    E = Wr.shape[1]
    H = W1.shape[2]
    x_flat = x.reshape(-1, D)
    T = x_flat.shape[0]
    SCHED = (T * TOPK // BT + E) * BT
    POSN = TOPK * T + 64
    nblk = SCHED // BT

    i1, i2, g1, g2 = _router(x_flat, Wr, br, min(512, T))
    ids_all = jnp.concatenate([i1.reshape(-1), i2.reshape(-1)])
    gts_all = jnp.concatenate([g1.reshape(-1), g2.reshape(-1)])

    counts = _make_count(T)(ids_all)
    meta, pos, block_expert = _make_dispatch(T, E, SCHED, POSN)(
        ids_all, gts_all, counts)
    xg = _make_gather(T, D, SCHED)(meta, x_flat)
    y = _gemm(xg, W1.astype(jnp.bfloat16), b1, W2.astype(jnp.bfloat16), b2,
              meta[:SCHED], block_expert[:nblk], nblk)
    out = _make_combine(T, D, SCHED)(pos, y)
    return out.reshape(x.shape)


# speedup vs baseline: 15.1039x; 1.5289x over previous
import functools

import jax
import jax.numpy as jnp
from jax import lax
from jax.experimental import pallas as pl
from jax.experimental.pallas import tpu as pltpu
from jax.experimental.pallas import tpu_sc as plsc

TOPK = 2
BT = 256
NLANE = 16
CHF = 48
CAPF = 2064


def _router_body(x_ref, wr_ref, br_ref, i1_ref, i2_ref, g1_ref, g2_ref):
    xb = x_ref[...]
    logits = jnp.dot(xb, wr_ref[...], preferred_element_type=jnp.float32)
    logits = logits + br_ref[...]
    bt, e = logits.shape
    iota_e = lax.broadcasted_iota(jnp.int32, (bt, e), 1)
    m1 = jnp.max(logits, axis=-1, keepdims=True)
    i1 = jnp.min(jnp.where(logits == m1, iota_e, e), axis=-1, keepdims=True)
    l2 = jnp.where(iota_e == i1, -jnp.inf, logits)
    m2 = jnp.max(l2, axis=-1, keepdims=True)
    i2 = jnp.min(jnp.where(l2 == m2, iota_e, e), axis=-1, keepdims=True)
    g1 = 1.0 / (1.0 + jnp.exp(m2 - m1))
    i1_ref[...] = i1
    i2_ref[...] = i2
    g1_ref[...] = g1
    g2_ref[...] = 1.0 - g1


def _router(x_flat, Wr, br, bt):
    T, D = x_flat.shape
    E = Wr.shape[1]
    o = jax.ShapeDtypeStruct((T, 1), jnp.int32)
    og = jax.ShapeDtypeStruct((T, 1), jnp.float32)
    return pl.pallas_call(
        _router_body,
        grid=(T // bt,),
        in_specs=[
            pl.BlockSpec((bt, D), lambda t: (t, 0)),
            pl.BlockSpec((D, E), lambda t: (0, 0)),
            pl.BlockSpec((1, E), lambda t: (0, 0)),
        ],
        out_specs=[pl.BlockSpec((bt, 1), lambda t: (t, 0))] * 4,
        out_shape=[o, o, og, og],
    )(x_flat, Wr, br.reshape(1, E))


def _wid():
    return lax.axis_index("s") * 2 + lax.axis_index("c")


def _stream_bases(cnt_v, E):
    cw = [cnt_v[i, pl.ds(0, NLANE)][0] for i in range(4 * E)]
    sp = [((c + CHF - 1) // CHF) * CHF for c in cw]
    ep = []
    for i in range(E):
        se = sp[4 * i] + sp[4 * i + 1] + sp[4 * i + 2] + sp[4 * i + 3]
        ep.append(((se + BT - 1) // BT) * BT)
    eb = []
    run = 0
    for i in range(E):
        eb.append(run)
        run = run + ep[i]
    sb = []
    for i in range(E):
        acc = eb[i]
        for kk in range(4):
            sb.append(acc)
            acc = acc + sp[4 * i + kk]
    return cw, eb, sb


def _make_count(T):
    Th = T // 2
    mesh = plsc.VectorSubcoreMesh(core_axis_name="c", subcore_axis_name="s")

    @functools.partial(
        pl.kernel,
        out_type=jax.ShapeDtypeStruct((32, NLANE), jnp.int32),
        mesh=mesh,
        compiler_params=pltpu.CompilerParams(
            needs_layout_passes=False, use_tc_tiling_on_sc=False),
        scratch_types=[
            pltpu.VMEM((Th,), jnp.int32),
            pltpu.VMEM((NLANE,), jnp.int32),
        ],
    )
    def count_k(ids_hbm, counts_hbm, ids_v, cnt_v):
        w = _wid()
        e = w // 4
        slot = (w // 2) % 2
        h = w % 2
        base = slot * T + h * Th
        pltpu.sync_copy(ids_hbm.at[pl.ds(base, Th)], ids_v)

        one = jnp.ones((NLANE,), jnp.int32)
        zero = jnp.zeros((NLANE,), jnp.int32)

        def step(i, acc):
            v = ids_v[pl.ds(i * NLANE, NLANE)]
            return acc + jnp.where(v == e, one, zero)

        acc = lax.fori_loop(0, Th // NLANE, step,
                            jnp.zeros((NLANE,), jnp.int32))
        total = jnp.sum(acc)
        cnt_v[...] = jnp.full((NLANE,), total, jnp.int32)
        pltpu.sync_copy(cnt_v, counts_hbm.at[w])

    return count_k


def _make_fused(T, E, D, SCHED):
    Th = T // 2
    NV = Th // NLANE
    MAXCH = CAPF // CHF
    DUMP = Th
    mesh = plsc.VectorSubcoreMesh(core_axis_name="c", subcore_axis_name="s")

    @functools.partial(
        pl.kernel,
        out_type=[
            jax.ShapeDtypeStruct((SCHED, D), jnp.float32),
            jax.ShapeDtypeStruct((SCHED, NLANE), jnp.float32),
            jax.ShapeDtypeStruct((64,), jnp.int32),
        ],
        mesh=mesh,
        compiler_params=pltpu.CompilerParams(
            needs_layout_passes=False, use_tc_tiling_on_sc=False),
        scratch_types=[
            pltpu.VMEM((Th,), jnp.int32),
            pltpu.VMEM((Th,), jnp.float32),
            pltpu.VMEM((4 * 8, NLANE), jnp.int32),
            pltpu.VMEM((CAPF,), jnp.int32),
            pltpu.VMEM((CAPF, NLANE), jnp.float32),
            pltpu.VMEM((CHF, D), jnp.float32),
            pltpu.VMEM((64,), jnp.int32),
            pltpu.SemaphoreType.DMA,
            pltpu.SemaphoreType.DMA,
        ],
    )
    def fused_k(ids_hbm, gts_hbm, counts_hbm, x_hbm,
                xg_hbm, gate_hbm, be_hbm,
                ids_v, gts_v, cnt_v, tok_c, gate_c, rows_v, be_v,
                gsem, wsem):
        w = _wid()
        e = w // 4
        slot = (w // 2) % 2
        h = w % 2
        base = slot * T + h * Th
        pltpu.sync_copy(ids_hbm.at[pl.ds(base, Th)], ids_v)
        pltpu.sync_copy(gts_hbm.at[pl.ds(base, Th)], gts_v)
        pltpu.sync_copy(counts_hbm, cnt_v)

        cw, eb, sb = _stream_bases(cnt_v, E)
        sb_w = jnp.int32(0)
        count_w = jnp.int32(0)
        for i in range(4 * E):
            sb_w = sb_w + jnp.where(w == i, sb[i], 0)
            count_w = count_w + jnp.where(w == i, cw[i], 0)
        nch_w = (count_w + CHF - 1) // CHF

        @pl.when(w == 0)
        def _():
            onev0 = jnp.ones((NLANE,), jnp.int32)
            zerov0 = jnp.zeros((NLANE,), jnp.int32)
            for j4 in range(4):
                lanes = lax.iota(jnp.int32, NLANE) + j4 * NLANE
                bev = jnp.full((NLANE,), -1, jnp.int32)
                for i in range(E):
                    bev = bev + jnp.where(
                        lanes >= jnp.full((NLANE,), eb[i] // BT, jnp.int32),
                        onev0, zerov0)
                be_v[pl.ds(j4 * NLANE, NLANE)] = bev
            pltpu.sync_copy(be_v, be_hbm)

        iota16 = lax.iota(jnp.int32, NLANE)
        onev = jnp.ones((NLANE,), jnp.int32)
        zerov = jnp.zeros((NLANE,), jnp.int32)
        dumpv = jnp.full((NLANE,), DUMP, jnp.int32)

        def step(iv, running):
            ids16 = ids_v[pl.ds(iv * NLANE, NLANE)]
            g16 = gts_v[pl.ds(iv * NLANE, NLANE)]
            m = ids16 == e
            mi = jnp.where(m, onev, zerov)
            pref = plsc.cumsum(mi)
            p = running + pref - 1
            idxs = jnp.where(m, p, dumpv)
            tvec = h * Th + iv * NLANE + iota16
            plsc.store_scatter(tok_c, [idxs], tvec)
            plsc.store_scatter(gate_c, [idxs, zerov], g16)
            return running + pref[NLANE - 1]

        lax.fori_loop(0, NV, step, jnp.int32(0))

        lo = jnp.zeros((NLANE,), jnp.int32)
        hi = jnp.full((NLANE,), T - 1, jnp.int32)

        def clamp_step(i, _):
            v = tok_c[pl.ds(i * NLANE, NLANE)]
            tok_c[pl.ds(i * NLANE, NLANE)] = jnp.minimum(
                jnp.maximum(v, lo), hi)
            return 0

        lax.fori_loop(0, CAPF // NLANE, clamp_step, 0)

        for j in range(MAXCH):
            @pl.when(j < nch_w)
            def _():
                row0 = sb_w + j * CHF
                pltpu.async_copy(
                    x_hbm.at[tok_c.at[pl.ds(j * CHF, CHF)]], rows_v,
                    gsem).wait()
                pltpu.async_copy(
                    rows_v, xg_hbm.at[pl.ds(row0, CHF)], wsem).wait()
                pltpu.sync_copy(gate_c.at[pl.ds(j * CHF, CHF)],
                                gate_hbm.at[pl.ds(row0, CHF)])

    return fused_k


def _gemm_body(be_ref, x_ref, w1_ref, b1_ref, w2_ref, b2_ref, meta_ref,
               y_ref, acc_ref, *, n_hc, bh):
    xb = x_ref[...].astype(jnp.bfloat16)
    acc_ref[...] = jnp.zeros_like(acc_ref)
    for hc in range(n_hc):
        w1c = w1_ref[0, :, hc * bh:(hc + 1) * bh]
        h = jnp.dot(xb, w1c, preferred_element_type=jnp.float32)
        h = h + b1_ref[0, :, hc * bh:(hc + 1) * bh]
        h = h * (1.0 / (1.0 + jnp.exp(-h)))
        w2c = w2_ref[0, hc * bh:(hc + 1) * bh, :]
        acc_ref[...] += jnp.dot(h.astype(jnp.bfloat16), w2c,
                                preferred_element_type=jnp.float32)
    g = meta_ref[...][:, 0:1]
    y_ref[...] = (acc_ref[...] + b2_ref[0]) * g


def _gemm(xg, W1b, b1, W2b, b2, meta, block_expert, nblk):
    _, D = xg.shape
    E, _, H = W1b.shape
    n_hc = 4
    grid_spec = pltpu.PrefetchScalarGridSpec(
        num_scalar_prefetch=1,
        grid=(nblk,),
        in_specs=[
            pl.BlockSpec((BT, D), lambda b, be: (b, 0)),
            pl.BlockSpec((1, D, H), lambda b, be: (be[b], 0, 0)),
            pl.BlockSpec((1, 1, H), lambda b, be: (be[b], 0, 0)),
            pl.BlockSpec((1, H, D), lambda b, be: (be[b], 0, 0)),
            pl.BlockSpec((1, 1, D), lambda b, be: (be[b], 0, 0)),
            pl.BlockSpec((BT, NLANE), lambda b, be: (b, 0)),
        ],
        out_specs=pl.BlockSpec((BT, D), lambda b, be: (b, 0)),
        scratch_shapes=[pltpu.VMEM((BT, D), jnp.float32)],
    )
    return pl.pallas_call(
        functools.partial(_gemm_body, n_hc=n_hc, bh=H // n_hc),
        grid_spec=grid_spec,
        out_shape=jax.ShapeDtypeStruct((nblk * BT, D), jnp.float32),
        compiler_params=pltpu.CompilerParams(vmem_limit_bytes=61_000_000),
    )(block_expert, xg, W1b, b1.reshape(E, 1, H), W2b, b2.reshape(E, 1, D),
      meta)


def _make_combine(T, E, D, SCHED):
    Th = T // 2
    tok_per = T // 32
    CH = 32
    nch = tok_per // CH
    mesh = plsc.VectorSubcoreMesh(core_axis_name="c", subcore_axis_name="s")

    @functools.partial(
        pl.kernel,
        out_type=jax.ShapeDtypeStruct((T, D), jnp.float32),
        mesh=mesh,
        compiler_params=pltpu.CompilerParams(
            needs_layout_passes=False, use_tc_tiling_on_sc=False),
        scratch_types=[
            pltpu.VMEM((Th,), jnp.int32),
            pltpu.VMEM((Th,), jnp.int32),
            pltpu.VMEM((4 * 8, NLANE), jnp.int32),
            pltpu.VMEM((T // 32,), jnp.int32),
            pltpu.VMEM((T // 32,), jnp.int32),
            pltpu.VMEM((32, D), jnp.float32),
            pltpu.VMEM((32, D), jnp.float32),
            pltpu.SemaphoreType.DMA,
            pltpu.SemaphoreType.DMA,
        ],
    )
    def combine_k(ids_hbm, counts_hbm, y_hbm, out_hbm,
                  ids0_v, ids1_v, cnt_v, p0_v, p1_v, a_v, b_v, sem0, sem1):
        w = _wid()
        h = w // 16
        pltpu.sync_copy(ids_hbm.at[pl.ds(h * Th, Th)], ids0_v)
        pltpu.sync_copy(ids_hbm.at[pl.ds(T + h * Th, Th)], ids1_v)
        pltpu.sync_copy(counts_hbm, cnt_v)
        cw, eb, sb = _stream_bases(cnt_v, E)

        onev = jnp.ones((NLANE,), jnp.int32)
        zerov = jnp.zeros((NLANE,), jnp.int32)
        lo = jnp.zeros((NLANE,), jnp.int32)
        hi = jnp.full((NLANE,), SCHED - 1, jnp.int32)
        npre_v = ((w % 16) * tok_per) // NLANE

        for s, (idsv, pv) in enumerate(((ids0_v, p0_v), (ids1_v, p1_v))):
            def pre_step(i, accs):
                v = idsv[pl.ds(i * NLANE, NLANE)]
                return tuple(
                    accs[i2] + jnp.where(v == i2, onev, zerov)
                    for i2 in range(E))

            accs = lax.fori_loop(
                0, npre_v, pre_step,
                tuple(jnp.zeros((NLANE,), jnp.int32) for _ in range(E)))
            C = [jnp.sum(a) for a in accs]
            for k in range(tok_per // NLANE):
                off = (npre_v + k) * NLANE
                v = idsv[pl.ds(off, NLANE)]
                rank = jnp.zeros((NLANE,), jnp.int32)
                sbv = jnp.zeros((NLANE,), jnp.int32)
                for i in range(E):
                    m = v == i
                    pr = plsc.cumsum(jnp.where(m, onev, zerov))
                    rank = jnp.where(m, C[i] + pr - 1, rank)
                    C[i] = C[i] + pr[NLANE - 1]
                    sb_i = jnp.where(h == 0, sb[4 * i + 2 * s],
                                     sb[4 * i + 2 * s + 1])
                    sbv = jnp.where(
                        m, jnp.full((NLANE,), sb_i, jnp.int32), sbv)
                pos = jnp.minimum(jnp.maximum(sbv + rank, lo), hi)
                pv[pl.ds(k * NLANE, NLANE)] = pos

        for j in range(nch):
            tbase = w * tok_per + j * CH
            c0 = pltpu.async_copy(
                y_hbm.at[p0_v.at[pl.ds(j * CH, CH)]], a_v, sem0)
            c1 = pltpu.async_copy(
                y_hbm.at[p1_v.at[pl.ds(j * CH, CH)]], b_v, sem1)
            c0.wait()
            c1.wait()
            for r in range(CH):
                def add_step(i, _):
                    sl = pl.ds(i * NLANE, NLANE)
                    a_v[r, sl] = a_v[r, sl] + b_v[r, sl]
                    return 0
                lax.fori_loop(0, D // NLANE, add_step, 0)
            pltpu.sync_copy(a_v, out_hbm.at[pl.ds(tbase, CH)])

    return combine_k


@jax.jit
def kernel(x, Wr, br, W1, b1, W2, b2):
    B, S, D = x.shape
    E = Wr.shape[1]
    x_flat = x.reshape(-1, D)
    T = x_flat.shape[0]
    worst = T * TOPK + 32 * (CHF - 1) + E * (BT - 1)
    nblk = (worst + BT - 1) // BT
    SCHED = nblk * BT

    i1, i2, g1, g2 = _router(x_flat, Wr, br, min(512, T))
    ids_all = jnp.concatenate([i1.reshape(-1), i2.reshape(-1)])
    gts_all = jnp.concatenate([g1.reshape(-1), g2.reshape(-1)])

    counts = _make_count(T)(ids_all)
    xg, gate2d, block_expert = _make_fused(T, E, D, SCHED)(
        ids_all, gts_all, counts, x_flat)
    y = _gemm(xg, W1.astype(jnp.bfloat16), b1, W2.astype(jnp.bfloat16), b2,
              gate2d, block_expert[:nblk], nblk)
    out = _make_combine(T, E, D, SCHED)(ids_all, counts, y)
    return out.reshape(x.shape)


# speedup vs baseline: 18.4540x; 1.2218x over previous
import functools

import jax
import jax.numpy as jnp
from jax import lax
from jax.experimental import pallas as pl
from jax.experimental.pallas import tpu as pltpu
from jax.experimental.pallas import tpu_sc as plsc

TOPK = 2
BT = 256
NLANE = 16
CHF = 48
CAPF = 2064


def _router_body(x_ref, wr_ref, br_ref, i1_ref, i2_ref, g1_ref, g2_ref):
    xb = x_ref[...]
    logits = jnp.dot(xb, wr_ref[...], preferred_element_type=jnp.float32)
    logits = logits + br_ref[...]
    bt, e = logits.shape
    iota_e = lax.broadcasted_iota(jnp.int32, (bt, e), 1)
    m1 = jnp.max(logits, axis=-1, keepdims=True)
    i1 = jnp.min(jnp.where(logits == m1, iota_e, e), axis=-1, keepdims=True)
    l2 = jnp.where(iota_e == i1, -jnp.inf, logits)
    m2 = jnp.max(l2, axis=-1, keepdims=True)
    i2 = jnp.min(jnp.where(l2 == m2, iota_e, e), axis=-1, keepdims=True)
    g1 = 1.0 / (1.0 + jnp.exp(m2 - m1))
    i1_ref[...] = i1
    i2_ref[...] = i2
    g1_ref[...] = g1
    g2_ref[...] = 1.0 - g1


def _router(x_flat, Wr, br, bt):
    T, D = x_flat.shape
    E = Wr.shape[1]
    o = jax.ShapeDtypeStruct((T, 1), jnp.int32)
    og = jax.ShapeDtypeStruct((T, 1), jnp.float32)
    return pl.pallas_call(
        _router_body,
        grid=(T // bt,),
        in_specs=[
            pl.BlockSpec((bt, D), lambda t: (t, 0)),
            pl.BlockSpec((D, E), lambda t: (0, 0)),
            pl.BlockSpec((1, E), lambda t: (0, 0)),
        ],
        out_specs=[pl.BlockSpec((bt, 1), lambda t: (t, 0))] * 4,
        out_shape=[o, o, og, og],
    )(x_flat, Wr, br.reshape(1, E))


def _wid():
    return lax.axis_index("s") * 2 + lax.axis_index("c")


def _stream_bases(cnt_v, E):
    cw = [cnt_v[i, pl.ds(0, NLANE)][0] for i in range(4 * E)]
    sp = [((c + CHF - 1) // CHF) * CHF for c in cw]
    ep = []
    for i in range(E):
        se = sp[4 * i] + sp[4 * i + 1] + sp[4 * i + 2] + sp[4 * i + 3]
        ep.append(((se + BT - 1) // BT) * BT)
    eb = []
    run = 0
    for i in range(E):
        eb.append(run)
        run = run + ep[i]
    sb = []
    for i in range(E):
        acc = eb[i]
        for kk in range(4):
            sb.append(acc)
            acc = acc + sp[4 * i + kk]
    return cw, eb, sb


def _make_count(T):
    Th = T // 2
    mesh = plsc.VectorSubcoreMesh(core_axis_name="c", subcore_axis_name="s")

    @functools.partial(
        pl.kernel,
        out_type=jax.ShapeDtypeStruct((32, NLANE), jnp.int32),
        mesh=mesh,
        compiler_params=pltpu.CompilerParams(
            needs_layout_passes=False, use_tc_tiling_on_sc=False),
        scratch_types=[
            pltpu.VMEM((Th,), jnp.int32),
            pltpu.VMEM((NLANE,), jnp.int32),
        ],
    )
    def count_k(ids_hbm, counts_hbm, ids_v, cnt_v):
        w = _wid()
        e = w // 4
        slot = (w // 2) % 2
        h = w % 2
        base = slot * T + h * Th
        pltpu.sync_copy(ids_hbm.at[pl.ds(base, Th)], ids_v)

        one = jnp.ones((NLANE,), jnp.int32)
        zero = jnp.zeros((NLANE,), jnp.int32)

        def step(i, acc):
            v = ids_v[pl.ds(i * NLANE, NLANE)]
            return acc + jnp.where(v == e, one, zero)

        acc = lax.fori_loop(0, Th // NLANE, step,
                            jnp.zeros((NLANE,), jnp.int32))
        total = jnp.sum(acc)
        cnt_v[...] = jnp.full((NLANE,), total, jnp.int32)
        pltpu.sync_copy(cnt_v, counts_hbm.at[w])

    return count_k


def _make_fused(T, E, D, SCHED):
    Th = T // 2
    NV = Th // NLANE
    MAXCH = CAPF // CHF
    DUMP = Th
    mesh = plsc.VectorSubcoreMesh(core_axis_name="c", subcore_axis_name="s")

    @functools.partial(
        pl.kernel,
        out_type=[
            jax.ShapeDtypeStruct((SCHED, D), jnp.float32),
            jax.ShapeDtypeStruct((64,), jnp.int32),
        ],
        mesh=mesh,
        compiler_params=pltpu.CompilerParams(needs_layout_passes=False),
        scratch_types=[
            pltpu.VMEM((Th,), jnp.int32),
            pltpu.VMEM((4 * 8, NLANE), jnp.int32),
            pltpu.VMEM((CAPF,), jnp.int32),
            pltpu.VMEM((CHF, D), jnp.float32),
            pltpu.VMEM((64,), jnp.int32),
            pltpu.SemaphoreType.DMA,
            pltpu.SemaphoreType.DMA,
        ],
    )
    def fused_k(ids_hbm, counts_hbm, x_hbm,
                xg_hbm, be_hbm,
                ids_v, cnt_v, tok_c, rows_v, be_v,
                gsem, wsem):
        w = _wid()
        e = w // 4
        slot = (w // 2) % 2
        h = w % 2
        base = pl.multiple_of(slot * T + h * Th, 8)
        pltpu.sync_copy(ids_hbm.at[pl.ds(base, Th)], ids_v)
        pltpu.sync_copy(counts_hbm, cnt_v)

        cw, eb, sb = _stream_bases(cnt_v, E)
        sb_w = jnp.int32(0)
        count_w = jnp.int32(0)
        for i in range(4 * E):
            sb_w = sb_w + jnp.where(w == i, sb[i], 0)
            count_w = count_w + jnp.where(w == i, cw[i], 0)
        nch_w = (count_w + CHF - 1) // CHF

        @pl.when(w == 0)
        def _():
            onev0 = jnp.ones((NLANE,), jnp.int32)
            zerov0 = jnp.zeros((NLANE,), jnp.int32)
            for j4 in range(4):
                lanes = lax.iota(jnp.int32, NLANE) + j4 * NLANE
                bev = jnp.full((NLANE,), -1, jnp.int32)
                for i in range(E):
                    bev = bev + jnp.where(
                        lanes >= jnp.full((NLANE,), eb[i] // BT, jnp.int32),
                        onev0, zerov0)
                be_v[pl.ds(j4 * NLANE, NLANE)] = bev
            pltpu.sync_copy(be_v, be_hbm)

        iota16 = lax.iota(jnp.int32, NLANE)
        onev = jnp.ones((NLANE,), jnp.int32)
        zerov = jnp.zeros((NLANE,), jnp.int32)
        dumpv = jnp.full((NLANE,), DUMP, jnp.int32)

        def step(iv, running):
            ids16 = ids_v[pl.ds(iv * NLANE, NLANE)]
            m = ids16 == e
            mi = jnp.where(m, onev, zerov)
            pref = plsc.cumsum(mi)
            p = running + pref - 1
            idxs = jnp.where(m, p, dumpv)
            tvec = h * Th + iv * NLANE + iota16
            plsc.store_scatter(tok_c, [idxs], tvec)
            return running + pref[NLANE - 1]

        lax.fori_loop(0, NV, step, jnp.int32(0))

        lo = jnp.zeros((NLANE,), jnp.int32)
        hi = jnp.full((NLANE,), T - 1, jnp.int32)

        def clamp_step(i, _):
            v = tok_c[pl.ds(i * NLANE, NLANE)]
            tok_c[pl.ds(i * NLANE, NLANE)] = jnp.minimum(
                jnp.maximum(v, lo), hi)
            return 0

        lax.fori_loop(0, CAPF // NLANE, clamp_step, 0)

        for j in range(MAXCH):
            @pl.when(j < nch_w)
            def _():
                row0 = pl.multiple_of(sb_w + j * CHF, 16)
                pltpu.async_copy(
                    x_hbm.at[tok_c.at[pl.ds(j * CHF, CHF)]], rows_v,
                    gsem).wait()
                pltpu.async_copy(
                    rows_v, xg_hbm.at[pl.ds(row0, CHF)], wsem).wait()

    return fused_k


def _gemm_body(be_ref, x_ref, w1_ref, b1_ref, w2_ref, b2_ref,
               y_ref, acc_ref, *, n_hc, bh):
    xb = x_ref[...].astype(jnp.bfloat16)
    acc_ref[...] = jnp.zeros_like(acc_ref)
    for hc in range(n_hc):
        w1c = w1_ref[0, :, hc * bh:(hc + 1) * bh]
        h = jnp.dot(xb, w1c, preferred_element_type=jnp.float32)
        h = h + b1_ref[0, :, hc * bh:(hc + 1) * bh]
        h = h * (1.0 / (1.0 + jnp.exp(-h)))
        w2c = w2_ref[0, hc * bh:(hc + 1) * bh, :]
        acc_ref[...] += jnp.dot(h.astype(jnp.bfloat16), w2c,
                                preferred_element_type=jnp.float32)
    y_ref[...] = acc_ref[...] + b2_ref[0]


def _gemm(xg, W1b, b1, W2b, b2, block_expert, nblk):
    _, D = xg.shape
    E, _, H = W1b.shape
    n_hc = 2
    grid_spec = pltpu.PrefetchScalarGridSpec(
        num_scalar_prefetch=1,
        grid=(nblk,),
        in_specs=[
            pl.BlockSpec((BT, D), lambda b, be: (b, 0)),
            pl.BlockSpec((1, D, H), lambda b, be: (be[b], 0, 0)),
            pl.BlockSpec((1, 1, H), lambda b, be: (be[b], 0, 0)),
            pl.BlockSpec((1, H, D), lambda b, be: (be[b], 0, 0)),
            pl.BlockSpec((1, 1, D), lambda b, be: (be[b], 0, 0)),
        ],
        out_specs=pl.BlockSpec((BT, D), lambda b, be: (b, 0)),
        scratch_shapes=[pltpu.VMEM((BT, D), jnp.float32)],
    )
    return pl.pallas_call(
        functools.partial(_gemm_body, n_hc=n_hc, bh=H // n_hc),
        grid_spec=grid_spec,
        out_shape=jax.ShapeDtypeStruct((nblk * BT, D), jnp.float32),
        compiler_params=pltpu.CompilerParams(vmem_limit_bytes=61_000_000),
    )(block_expert, xg, W1b, b1.reshape(E, 1, H), W2b, b2.reshape(E, 1, D))


def _make_combine(T, E, D, SCHED):
    Th = T // 2
    tok_per = T // 32
    CH = 32
    nch = tok_per // CH
    mesh = plsc.VectorSubcoreMesh(core_axis_name="c", subcore_axis_name="s")

    @functools.partial(
        pl.kernel,
        out_type=jax.ShapeDtypeStruct((T, D), jnp.float32),
        mesh=mesh,
        compiler_params=pltpu.CompilerParams(needs_layout_passes=False),
        scratch_types=[
            pltpu.VMEM((Th,), jnp.int32),
            pltpu.VMEM((Th,), jnp.int32),
            pltpu.VMEM((4 * 8, NLANE), jnp.int32),
            pltpu.VMEM((T // 32,), jnp.int32),
            pltpu.VMEM((T // 32,), jnp.int32),
            pltpu.VMEM((T // 32,), jnp.float32),
            pltpu.VMEM((T // 32,), jnp.float32),
            pltpu.VMEM((32, D), jnp.float32),
            pltpu.VMEM((32, D), jnp.float32),
            pltpu.SemaphoreType.DMA,
            pltpu.SemaphoreType.DMA,
        ],
    )
    def combine_k(ids_hbm, gts_hbm, counts_hbm, y_hbm, out_hbm,
                  ids0_v, ids1_v, cnt_v, p0_v, p1_v, g0_v, g1_v,
                  a_v, b_v, sem0, sem1):
        w = _wid()
        h = w // 16
        t0 = w * tok_per
        pltpu.sync_copy(ids_hbm.at[pl.ds(pl.multiple_of(h * Th, 8), Th)],
                        ids0_v)
        pltpu.sync_copy(ids_hbm.at[pl.ds(pl.multiple_of(T + h * Th, 8), Th)],
                        ids1_v)
        pltpu.sync_copy(gts_hbm.at[pl.ds(pl.multiple_of(t0, 8), tok_per)],
                        g0_v)
        pltpu.sync_copy(gts_hbm.at[pl.ds(pl.multiple_of(T + t0, 8),
                                         tok_per)], g1_v)
        pltpu.sync_copy(counts_hbm, cnt_v)
        cw, eb, sb = _stream_bases(cnt_v, E)

        onev = jnp.ones((NLANE,), jnp.int32)
        zerov = jnp.zeros((NLANE,), jnp.int32)
        lo = jnp.zeros((NLANE,), jnp.int32)
        hi = jnp.full((NLANE,), SCHED - 1, jnp.int32)
        npre_v = ((w % 16) * tok_per) // NLANE

        for s, (idsv, pv) in enumerate(((ids0_v, p0_v), (ids1_v, p1_v))):
            def pre_step(i, accs):
                v = idsv[pl.ds(i * NLANE, NLANE)]
                return tuple(
                    accs[i2] + jnp.where(v == i2, onev, zerov)
                    for i2 in range(E))

            accs = lax.fori_loop(
                0, npre_v, pre_step,
                tuple(jnp.zeros((NLANE,), jnp.int32) for _ in range(E)))
            C = [jnp.sum(a) for a in accs]
            for k in range(tok_per // NLANE):
                off = (npre_v + k) * NLANE
                v = idsv[pl.ds(off, NLANE)]
                rank = jnp.zeros((NLANE,), jnp.int32)
                sbv = jnp.zeros((NLANE,), jnp.int32)
                for i in range(E):
                    m = v == i
                    pr = plsc.cumsum(jnp.where(m, onev, zerov))
                    rank = jnp.where(m, C[i] + pr - 1, rank)
                    C[i] = C[i] + pr[NLANE - 1]
                    sb_i = jnp.where(h == 0, sb[4 * i + 2 * s],
                                     sb[4 * i + 2 * s + 1])
                    sbv = jnp.where(
                        m, jnp.full((NLANE,), sb_i, jnp.int32), sbv)
                pos = jnp.minimum(jnp.maximum(sbv + rank, lo), hi)
                pv[pl.ds(k * NLANE, NLANE)] = pos

        for j in range(nch):
            tbase = pl.multiple_of(w * tok_per + j * CH, 8)
            c0 = pltpu.async_copy(
                y_hbm.at[p0_v.at[pl.ds(j * CH, CH)]], a_v, sem0)
            c1 = pltpu.async_copy(
                y_hbm.at[p1_v.at[pl.ds(j * CH, CH)]], b_v, sem1)
            c0.wait()
            c1.wait()
            gvec0 = [g0_v[pl.ds(j * CH + q * NLANE, NLANE)]
                     for q in range(CH // NLANE)]
            gvec1 = [g1_v[pl.ds(j * CH + q * NLANE, NLANE)]
                     for q in range(CH // NLANE)]
            for r in range(CH):
                ga = gvec0[r // NLANE][r % NLANE]
                gb = gvec1[r // NLANE][r % NLANE]
                gav = jnp.full((NLANE,), ga, jnp.float32)
                gbv = jnp.full((NLANE,), gb, jnp.float32)

                def add_step(i, _):
                    sl = pl.ds(i * NLANE, NLANE)
                    a_v[r, sl] = a_v[r, sl] * gav + b_v[r, sl] * gbv
                    return 0
                lax.fori_loop(0, D // NLANE, add_step, 0)
            pltpu.sync_copy(a_v, out_hbm.at[pl.ds(tbase, CH)])

    return combine_k


@jax.jit
def kernel(x, Wr, br, W1, b1, W2, b2):
    B, S, D = x.shape
    E = Wr.shape[1]
    x_flat = x.reshape(-1, D)
    T = x_flat.shape[0]
    worst = T * TOPK + 32 * (CHF - 1) + E * (BT - 1)
    nblk = (worst + BT - 1) // BT
    SCHED = nblk * BT

    i1, i2, g1, g2 = _router(x_flat, Wr, br, min(512, T))
    ids_all = jnp.concatenate([i1.reshape(-1), i2.reshape(-1)])
    gts_all = jnp.concatenate([g1.reshape(-1), g2.reshape(-1)])

    counts = _make_count(T)(ids_all)
    xg, block_expert = _make_fused(T, E, D, SCHED)(
        ids_all, counts, x_flat)
    y = _gemm(xg, W1.astype(jnp.bfloat16), b1, W2.astype(jnp.bfloat16), b2,
              block_expert[:nblk], nblk)
    out = _make_combine(T, E, D, SCHED)(ids_all, gts_all, counts, y)
    return out.reshape(x.shape)
